# Initial kernel scaffold; baseline (speedup 1.0000x reference)
import jax
import jax.numpy as jnp
from jax.experimental import pallas as pl


def kernel(x, gate_w, W1, W2, W3, Ws1, Ws2, Ws3):
    raise NotImplementedError("write your pallas kernel here")


# speedup vs baseline: 1.1389x; 1.1389x over previous
import functools

import jax
import jax.numpy as jnp
from jax.experimental import pallas as pl
from jax.experimental.pallas import tpu as pltpu

N_TOK = 2048
DIM = 2048
INTER = 1024
E = 8

BT = 256
BI = 512
J = INTER // BI


def _routing_body(l_ref, w_ref):
    logits = l_ref[...]
    m = jnp.max(logits, axis=1, keepdims=True)
    p = jnp.exp(logits - m)
    p = p / jnp.sum(p, axis=1, keepdims=True)
    iot = jax.lax.broadcasted_iota(jnp.int32, p.shape, 1)
    m1 = jnp.max(p, axis=1, keepdims=True)
    i1 = jnp.min(jnp.where(p == m1, iot, E), axis=1, keepdims=True)
    p2 = jnp.where(iot == i1, -jnp.inf, p)
    m2 = jnp.max(p2, axis=1, keepdims=True)
    i2 = jnp.min(jnp.where(p2 == m2, iot, E), axis=1, keepdims=True)
    w_ref[...] = jnp.where(iot == i1, m1, 0.0) + jnp.where(iot == i2, m2, 0.0)


def _route(logits):
    return pl.pallas_call(
        _routing_body,
        grid=(N_TOK // BT,),
        in_specs=[
            pl.BlockSpec((BT, E), lambda t: (t, 0)),
        ],
        out_specs=pl.BlockSpec((BT, E), lambda t: (t, 0)),
        out_shape=jax.ShapeDtypeStruct((N_TOK, E), jnp.float32),
    )(logits)


def _moe_body(xb_ref, w_ref, w1_ref, w3_ref, w2_ref, out_ref, w1b, w3b, w2b):
    e = pl.program_id(0)
    j = pl.program_id(1)
    t = pl.program_id(2)

    @pl.when(t == 0)
    def _():
        w1b[...] = w1_ref[0].astype(jnp.bfloat16)
        w3b[...] = w3_ref[0].astype(jnp.bfloat16)
        w2b[...] = w2_ref[0].astype(jnp.bfloat16)

    xt = xb_ref[pl.ds(t * BT, BT), :]
    h1 = jax.lax.dot_general(xt, w1b[...], (((1,), (1,)), ((), ())),
                             preferred_element_type=jnp.float32)
    h3 = jax.lax.dot_general(xt, w3b[...], (((1,), (1,)), ((), ())),
                             preferred_element_type=jnp.float32)
    g = (jax.nn.silu(h1) * h3).astype(jnp.bfloat16)
    o = jax.lax.dot_general(g, w2b[...], (((1,), (1,)), ((), ())),
                            preferred_element_type=jnp.float32)
    wtok = w_ref[pl.ds(t * BT, BT), :]
    onehot = (jax.lax.broadcasted_iota(jnp.int32, (1, E), 1) == e)
    wcol = jnp.sum(jnp.where(onehot, wtok, 0.0), axis=1, keepdims=True)
    contrib = o * wcol
    first = jnp.logical_and(e == 0, j == 0)

    @pl.when(first)
    def _():
        out_ref[pl.ds(t * BT, BT), :] = contrib

    @pl.when(jnp.logical_not(first))
    def _():
        out_ref[pl.ds(t * BT, BT), :] += contrib


def _moe(xb, w, W1, W3, W2):
    ne = W1.shape[0]
    return pl.pallas_call(
        _moe_body,
        grid=(ne, J, N_TOK // BT),
        in_specs=[
            pl.BlockSpec((N_TOK, DIM), lambda e, j, t: (0, 0)),
            pl.BlockSpec((N_TOK, E), lambda e, j, t: (0, 0)),
            pl.BlockSpec((1, BI, DIM), lambda e, j, t: (e, j, 0)),
            pl.BlockSpec((1, BI, DIM), lambda e, j, t: (e, j, 0)),
            pl.BlockSpec((1, DIM, BI), lambda e, j, t: (e, 0, j)),
        ],
        out_specs=pl.BlockSpec((N_TOK, DIM), lambda e, j, t: (0, 0)),
        out_shape=jax.ShapeDtypeStruct((N_TOK, DIM), jnp.float32),
        scratch_shapes=[
            pltpu.VMEM((BI, DIM), jnp.bfloat16),
            pltpu.VMEM((BI, DIM), jnp.bfloat16),
            pltpu.VMEM((DIM, BI), jnp.bfloat16),
        ],
    )(xb, w, W1, W3, W2)


def kernel(x, gate_w, W1, W2, W3, Ws1, Ws2, Ws3):
    xb = x.astype(jnp.bfloat16)
    w = _route(x @ gate_w.T)
    y_moe = _moe(xb, w, W1, W3, W2)
    ones = jnp.ones((N_TOK, E), jnp.float32)
    y_sh = _moe(xb, ones, Ws1[None], Ws3[None], Ws2[None])
    return y_moe + y_sh
